# baseline (speedup 1.0000x reference)
import functools

import jax
import jax.numpy as jnp
from jax.experimental import pallas as pl
from jax.experimental.pallas import tpu as pltpu

_TK = 2048
_QC = 8
_CW = 128
_DH = 64


def _nn_tile_kernel(nq, tk, nqc, xt_ref, obst_ref, min_ref, idx_ref,
                    dist_scratch, absk_scratch):
    t = pl.program_id(0)
    qq = pl.program_id(1)

    @pl.when(qq == 0)
    def _():
        absk_scratch[...] = jnp.abs(obst_ref[...])

    xq8 = xt_ref[0]
    axq8 = jnp.maximum(jnp.abs(xq8), 1e-30)

    d = xq8.shape[0]
    xqs = [jax.lax.slice(xq8, (0, j), (d, j + 1)) for j in range(_QC)]
    axqs = [jax.lax.slice(axq8, (0, j), (d, j + 1)) for j in range(_QC)]

    def cbody(c, carry):
        lo = c * _CW
        keys_c = obst_ref[:, pl.ds(lo, _CW)]
        absk_c = absk_scratch[:, pl.ds(lo, _CW)]
        rows = []
        for j in range(_QC):
            num = jnp.abs(keys_c - xqs[j])
            den = absk_c + axqs[j]
            rows.append(jnp.sum(num / den, axis=0, keepdims=True))
        dist_scratch[pl.ds(qq * _QC, _QC), pl.ds(lo, _CW)] = (
            jnp.concatenate(rows, 0))
        return carry

    jax.lax.fori_loop(0, tk // _CW, cbody, 0)

    @pl.when(qq == nqc - 1)
    def _():
        dmat = dist_scratch[...]
        m = jnp.min(dmat, axis=1, keepdims=True)
        am = jnp.argmin(dmat, axis=1).astype(jnp.int32)[:, None]
        gi = am + t * tk

        @pl.when(t == 0)
        def _():
            min_ref[...] = m
            idx_ref[...] = gi

        @pl.when(t > 0)
        def _():
            old_m = min_ref[...]
            take = m < old_m
            min_ref[...] = jnp.where(take, m, old_m)
            idx_ref[...] = jnp.where(take, gi, idx_ref[...])


def kernel(x, observations):
    nq, d = x.shape
    k = observations.shape[0]
    tk = _TK
    ntiles = -(-k // tk)
    kpad = ntiles * tk
    nqc = nq // _QC

    obst = observations.T
    if kpad > k:
        pad = jnp.broadcast_to(obst[:, :1], (d, kpad - k))
        obst = jnp.concatenate([obst, pad], axis=1)
    xt3 = x.reshape(nqc, _QC, d).transpose(0, 2, 1)

    min2d, idx2d = pl.pallas_call(
        functools.partial(_nn_tile_kernel, nq, tk, nqc),
        grid=(ntiles, nqc),
        in_specs=[
            pl.BlockSpec((1, d, _QC), lambda t, q: (q, 0, 0)),
            pl.BlockSpec((d, tk), lambda t, q: (0, t)),
        ],
        out_specs=[
            pl.BlockSpec((nq, 1), lambda t, q: (0, 0)),
            pl.BlockSpec((nq, 1), lambda t, q: (0, 0)),
        ],
        out_shape=[
            jax.ShapeDtypeStruct((nq, 1), jnp.float32),
            jax.ShapeDtypeStruct((nq, 1), jnp.int32),
        ],
        scratch_shapes=[
            pltpu.VMEM((nq, tk), jnp.float32),
            pltpu.VMEM((d, tk), jnp.float32),
        ],
    )(xt3, obst)

    return min2d[:, 0], idx2d[:, 0]


# speedup vs baseline: 3.0958x; 3.0958x over previous
import functools

import jax
import jax.numpy as jnp
from jax.experimental import pallas as pl
from jax.experimental.pallas import tpu as pltpu

_TK = 2048
_QC = 64
_CW = 128
_DQ = 32


def _nn_tile_kernel(nq, tk, nqc, qc, xt_ref, obst_ref, min_ref, idx_ref,
                    dist_scratch, absk_scratch):
    t = pl.program_id(0)
    qq = pl.program_id(1)

    @pl.when(qq == 0)
    def _():
        absk_scratch[...] = jnp.abs(obst_ref[...])

    xq8 = xt_ref[0]
    axq8 = jnp.maximum(jnp.abs(xq8), 1e-30)

    d = xq8.shape[0]
    xqs = [jax.lax.slice(xq8, (0, j), (d, j + 1)) for j in range(qc)]
    axqs = [jax.lax.slice(axq8, (0, j), (d, j + 1)) for j in range(qc)]

    for c in range(tk // _CW):
        lo, hi = c * _CW, (c + 1) * _CW
        rows = []
        for j in range(qc):
            acc = None
            for h in range(0, d, _DQ):
                kv = obst_ref[h:h + _DQ, lo:hi]
                av = absk_scratch[h:h + _DQ, lo:hi]
                xqh = jax.lax.slice(xqs[j], (h, 0), (h + _DQ, 1))
                axqh = jax.lax.slice(axqs[j], (h, 0), (h + _DQ, 1))
                num = jnp.abs(kv - xqh)
                den = av + axqh
                part = jnp.sum((num / den).reshape(_DQ // 8, 8, _CW), axis=0)
                acc = part if acc is None else acc + part
            rows.append(jnp.sum(acc, axis=0, keepdims=True))
        dist_scratch[pl.ds(qq * qc, qc), lo:hi] = jnp.concatenate(rows, 0)

    @pl.when(qq == nqc - 1)
    def _():
        dmat = dist_scratch[...]
        m = jnp.min(dmat, axis=1, keepdims=True)
        am = jnp.argmin(dmat, axis=1).astype(jnp.int32)[:, None]
        gi = am + t * tk

        @pl.when(t == 0)
        def _():
            min_ref[...] = m
            idx_ref[...] = gi

        @pl.when(t > 0)
        def _():
            old_m = min_ref[...]
            take = m < old_m
            min_ref[...] = jnp.where(take, m, old_m)
            idx_ref[...] = jnp.where(take, gi, idx_ref[...])


def kernel(x, observations):
    nq, d = x.shape
    k = observations.shape[0]
    tk = _TK
    ntiles = -(-k // tk)
    kpad = ntiles * tk
    qc = min(_QC, nq)
    nqc = nq // qc

    obst = observations.T
    if kpad > k:
        pad = jnp.broadcast_to(obst[:, :1], (d, kpad - k))
        obst = jnp.concatenate([obst, pad], axis=1)
    xt3 = x.reshape(nqc, qc, d).transpose(0, 2, 1)

    min2d, idx2d = pl.pallas_call(
        functools.partial(_nn_tile_kernel, nq, tk, nqc, qc),
        grid=(ntiles, nqc),
        in_specs=[
            pl.BlockSpec((1, d, qc), lambda t, q: (q, 0, 0)),
            pl.BlockSpec((d, tk), lambda t, q: (0, t)),
        ],
        out_specs=[
            pl.BlockSpec((nq, 1), lambda t, q: (0, 0)),
            pl.BlockSpec((nq, 1), lambda t, q: (0, 0)),
        ],
        out_shape=[
            jax.ShapeDtypeStruct((nq, 1), jnp.float32),
            jax.ShapeDtypeStruct((nq, 1), jnp.int32),
        ],
        scratch_shapes=[
            pltpu.VMEM((nq, tk), jnp.float32),
            pltpu.VMEM((d, tk), jnp.float32),
        ],
    )(xt3, obst)

    return min2d[:, 0], idx2d[:, 0]


# speedup vs baseline: 4.5582x; 1.4724x over previous
import functools

import jax
import jax.numpy as jnp
import numpy as np
from jax.experimental import pallas as pl
from jax.experimental.pallas import tpu as pltpu
from jax.sharding import Mesh, PartitionSpec as P

def _shmap(f, mesh, in_specs, out_specs):
    try:
        return jax.shard_map(f, mesh=mesh, in_specs=in_specs,
                             out_specs=out_specs, check_vma=False)
    except (AttributeError, TypeError):
        from jax.experimental.shard_map import shard_map as _shard_map
        return _shard_map(f, mesh=mesh, in_specs=in_specs,
                          out_specs=out_specs, check_rep=False)

_TK = 2048
_QC = 64
_CW = 128
_DQ = 32


def _nn_tile_kernel(nq, tk, nqc, qc, xt_ref, obst_ref, min_ref, idx_ref,
                    dist_scratch, absk_scratch):
    t = pl.program_id(0)
    qq = pl.program_id(1)

    @pl.when(qq == 0)
    def _():
        absk_scratch[...] = jnp.abs(obst_ref[...])

    xq8 = xt_ref[0]
    axq8 = jnp.maximum(jnp.abs(xq8), 1e-30)

    d = xq8.shape[0]
    xqs = [jax.lax.slice(xq8, (0, j), (d, j + 1)) for j in range(qc)]
    axqs = [jax.lax.slice(axq8, (0, j), (d, j + 1)) for j in range(qc)]

    for c in range(tk // _CW):
        lo, hi = c * _CW, (c + 1) * _CW
        rows = []
        for j in range(qc):
            acc = None
            for h in range(0, d, _DQ):
                kv = obst_ref[h:h + _DQ, lo:hi]
                av = absk_scratch[h:h + _DQ, lo:hi]
                xqh = jax.lax.slice(xqs[j], (h, 0), (h + _DQ, 1))
                axqh = jax.lax.slice(axqs[j], (h, 0), (h + _DQ, 1))
                num = jnp.abs(kv - xqh)
                den = av + axqh
                part = jnp.sum((num / den).reshape(_DQ // 8, 8, _CW), axis=0)
                acc = part if acc is None else acc + part
            rows.append(jnp.sum(acc, axis=0, keepdims=True))
        dist_scratch[pl.ds(qq * qc, qc), lo:hi] = jnp.concatenate(rows, 0)

    @pl.when(qq == nqc - 1)
    def _():
        dmat = dist_scratch[...]
        m = jnp.min(dmat, axis=1, keepdims=True)
        am = jnp.argmin(dmat, axis=1).astype(jnp.int32)[:, None]
        gi = am + t * tk

        @pl.when(t == 0)
        def _():
            min_ref[...] = m
            idx_ref[...] = gi

        @pl.when(t > 0)
        def _():
            old_m = min_ref[...]
            take = m < old_m
            min_ref[...] = jnp.where(take, m, old_m)
            idx_ref[...] = jnp.where(take, gi, idx_ref[...])


def _nn_scan(x, observations):
    nq, d = x.shape
    k = observations.shape[0]
    tk = _TK
    ntiles = -(-k // tk)
    kpad = ntiles * tk
    qc = min(_QC, nq)
    nqc = nq // qc

    obst = observations.T
    if kpad > k:
        pad = jnp.broadcast_to(obst[:, :1], (d, kpad - k))
        obst = jnp.concatenate([obst, pad], axis=1)
    xt3 = x.reshape(nqc, qc, d).transpose(0, 2, 1)

    min2d, idx2d = pl.pallas_call(
        functools.partial(_nn_tile_kernel, nq, tk, nqc, qc),
        grid=(ntiles, nqc),
        in_specs=[
            pl.BlockSpec((1, d, qc), lambda t, q: (q, 0, 0)),
            pl.BlockSpec((d, tk), lambda t, q: (0, t)),
        ],
        out_specs=[
            pl.BlockSpec((nq, 1), lambda t, q: (0, 0)),
            pl.BlockSpec((nq, 1), lambda t, q: (0, 0)),
        ],
        out_shape=[
            jax.ShapeDtypeStruct((nq, 1), jnp.float32),
            jax.ShapeDtypeStruct((nq, 1), jnp.int32),
        ],
        scratch_shapes=[
            pltpu.VMEM((nq, tk), jnp.float32),
            pltpu.VMEM((d, tk), jnp.float32),
        ],
    )(xt3, obst)

    return min2d[:, 0], idx2d[:, 0]


def _local_lookup(x, obs_local):
    shard = jax.lax.axis_index("x")
    m, i = _nn_scan(x, obs_local)
    gi = i + shard.astype(jnp.int32) * obs_local.shape[0]
    return m[None, :], gi[None, :]


def kernel(x, observations):
    devs = jax.devices()
    if len(devs) < 2 or observations.shape[0] % 2 != 0:
        return _nn_scan(x, observations)

    mesh = Mesh(np.asarray(devs[:2]), ("x",))
    m2, i2 = _shmap(
        _local_lookup, mesh,
        in_specs=(P(), P("x", None)),
        out_specs=(P("x", None), P("x", None)),
    )(x, observations)
    take = m2[1] < m2[0]
    return jnp.where(take, m2[1], m2[0]), jnp.where(take, i2[1], i2[0])
